# baseline (speedup 1.0000x reference)
import functools

import jax
import jax.numpy as jnp
from jax import lax
from jax.experimental import pallas as pl
from jax.experimental.pallas import tpu as pltpu
from jax.experimental.pallas import tpu_sc as plsc

_B = 4
_S = 8192
_D = 1024
_NC, _NS, _L = 2, 16, 16
_NW = _NC * _NS
_ROWS = _S // _NW
_R = 16
_NJ = _ROWS // _R


def _sc_body(x_hbm, t_hbm, out_hbm,
             xb0, xb1, xb2, xb3, tb0, tb1,
             si0, si1, si2, si3, so0, so1, so2, so3, st0, st1):
    xbs = (xb0, xb1, xb2, xb3)
    sis = (si0, si1, si2, si3)
    sos = (so0, so1, so2, so3)
    tbs = (tb0, tb1)
    sts = (st0, st1)

    wid = lax.axis_index("s") * _NC + lax.axis_index("c")
    base = wid * _ROWS

    def start_in(j, batch, bufi):
        pltpu.async_copy(
            x_hbm.at[batch, pl.ds(base + j * _R, _R), :], xbs[bufi], sis[bufi])

    def wait_in(bufi):
        pltpu.make_async_copy(
            x_hbm.at[0, pl.ds(base, _R), :], xbs[bufi], sis[bufi]).wait()

    def start_t(j, ti):
        pltpu.async_copy(
            t_hbm.at[pl.ds(base + j * _R, _R), :], tbs[ti], sts[ti])

    def wait_t(ti):
        pltpu.make_async_copy(
            t_hbm.at[pl.ds(base, _R), :], tbs[ti], sts[ti]).wait()

    def start_out(j, batch, bufi):
        pltpu.async_copy(
            xbs[bufi], out_hbm.at[batch, pl.ds(base + j * _R, _R), :],
            sos[bufi])

    def wait_out(bufi):
        pltpu.make_async_copy(
            xbs[bufi], out_hbm.at[0, pl.ds(base, _R), :], sos[bufi]).wait()

    start_t(0, 0)
    start_in(0, 0, 0)
    start_in(0, 1, 1)

    def outer(g, carry):
        for m in range(8):
            jj, b = divmod(m, 4)
            j = 2 * g + jj
            wait_in(b)
            if b == 0:
                wait_t(jj)

            def _add(r, c3):
                for u in range(_D // _L):
                    s = pl.ds(u * _L, _L)
                    plsc.addupdate(xbs[b].at[r, s], tbs[jj][r, s])
                return c3

            lax.fori_loop(0, _R, _add, 0)
            start_out(j, b, b)
            if b == 0:
                if m == 0:
                    @pl.when(2 * g + 1 < _NJ)
                    def _():
                        start_t(j + 1, 1)
                else:
                    @pl.when(g < (_NJ // 2) - 1)
                    def _():
                        start_t(j + 1, 0)
            b2 = (m + 2) % 4
            if m < 2:
                @pl.when(g > 0)
                def _():
                    wait_out(b2)
            else:
                wait_out(b2)
            if m < 6:
                start_in(2 * g + (m + 2) // 4, b2, b2)
            else:
                @pl.when(g < (_NJ // 2) - 1)
                def _():
                    start_in(2 * (g + 1), b2, b2)
        return carry

    lax.fori_loop(0, _NJ // 2, outer, 0)
    wait_out(2)
    wait_out(3)


@jax.jit
def _sc_add(x, table):
    mesh = plsc.VectorSubcoreMesh(core_axis_name="c", subcore_axis_name="s")
    f = functools.partial(
        pl.kernel,
        mesh=mesh,
        out_type=jax.ShapeDtypeStruct((_B, _S, _D), jnp.float32),
        scratch_types=(
            [pltpu.VMEM((_R, _D), jnp.float32)] * 4
            + [pltpu.VMEM((_R, _D), jnp.float32)] * 2
            + [pltpu.SemaphoreType.DMA] * 10
        ),
    )(_sc_body)
    return f(x, table)


def kernel(x, table):
    return _sc_add(x, table)


# speedup vs baseline: 1.7881x; 1.7881x over previous
import functools

import jax
import jax.numpy as jnp
from jax import lax
from jax.experimental import pallas as pl
from jax.experimental.pallas import tpu as pltpu
from jax.experimental.pallas import tpu_sc as plsc

_B = 4
_S = 8192
_D = 1024
_NC, _NS, _L = 2, 16, 16
_NW = _NC * _NS
_ROWS = _S // _NW
_R = 16
_NJ = _ROWS // _R


def _sc_body(x_hbm, t_hbm, out_hbm,
             xb0, xb1, xb2, xb3, tb0, tb1,
             si0, si1, si2, si3, so0, so1, so2, so3, st0, st1):
    xbs = (xb0, xb1, xb2, xb3)
    sis = (si0, si1, si2, si3)
    sos = (so0, so1, so2, so3)
    tbs = (tb0, tb1)
    sts = (st0, st1)

    wid = lax.axis_index("s") * _NC + lax.axis_index("c")
    base = wid * _ROWS

    def start_in(j, batch, bufi):
        pltpu.async_copy(
            x_hbm.at[batch, pl.ds(base + j * _R, _R), :], xbs[bufi], sis[bufi])

    def wait_in(bufi):
        pltpu.make_async_copy(
            x_hbm.at[0, pl.ds(base, _R), :], xbs[bufi], sis[bufi]).wait()

    def start_t(j, ti):
        pltpu.async_copy(
            t_hbm.at[pl.ds(base + j * _R, _R), :], tbs[ti], sts[ti])

    def wait_t(ti):
        pltpu.make_async_copy(
            t_hbm.at[pl.ds(base, _R), :], tbs[ti], sts[ti]).wait()

    def start_out(j, batch, bufi):
        pltpu.async_copy(
            xbs[bufi], out_hbm.at[batch, pl.ds(base + j * _R, _R), :],
            sos[bufi])

    def wait_out(bufi):
        pltpu.make_async_copy(
            xbs[bufi], out_hbm.at[0, pl.ds(base, _R), :], sos[bufi]).wait()

    start_t(0, 0)
    start_in(0, 0, 0)
    start_in(0, 1, 1)

    def outer(g, carry):
        for m in range(8):
            jj, b = divmod(m, 4)
            j = 2 * g + jj
            wait_in(b)
            if b == 0:
                wait_t(jj)

            def _add(r, c3):
                for u in range(_D // _L):
                    s = pl.ds(u * _L, _L)
                    xbs[b][r, s] = xbs[b][r, s] + tbs[jj][r, s]
                return c3

            lax.fori_loop(0, _R, _add, 0)
            start_out(j, b, b)
            if b == 0:
                if m == 0:
                    @pl.when(2 * g + 1 < _NJ)
                    def _():
                        start_t(j + 1, 1)
                else:
                    @pl.when(g < (_NJ // 2) - 1)
                    def _():
                        start_t(j + 1, 0)
            b2 = (m + 2) % 4
            if m < 2:
                @pl.when(g > 0)
                def _():
                    wait_out(b2)
            else:
                wait_out(b2)
            if m < 6:
                start_in(2 * g + (m + 2) // 4, b2, b2)
            else:
                @pl.when(g < (_NJ // 2) - 1)
                def _():
                    start_in(2 * (g + 1), b2, b2)
        return carry

    lax.fori_loop(0, _NJ // 2, outer, 0)
    wait_out(2)
    wait_out(3)


@jax.jit
def _sc_add(x, table):
    mesh = plsc.VectorSubcoreMesh(core_axis_name="c", subcore_axis_name="s")
    f = functools.partial(
        pl.kernel,
        mesh=mesh,
        out_type=jax.ShapeDtypeStruct((_B, _S, _D), jnp.float32),
        scratch_types=(
            [pltpu.VMEM((_R, _D), jnp.float32)] * 4
            + [pltpu.VMEM((_R, _D), jnp.float32)] * 2
            + [pltpu.SemaphoreType.DMA] * 10
        ),
    )(_sc_body)
    return f(x, table)


def kernel(x, table):
    return _sc_add(x, table)


# speedup vs baseline: 2.0584x; 1.1512x over previous
import functools

import jax
import jax.numpy as jnp
from jax import lax
from jax.experimental import pallas as pl
from jax.experimental.pallas import tpu as pltpu
from jax.experimental.pallas import tpu_sc as plsc

_B = 4
_S = 8192
_D = 1024
_NC, _NS, _L = 2, 16, 16
_NW = _NC * _NS
_ROWS = _S // _NW
_R = 8
_NJ = _ROWS // _R
_ST = 3
_NG = 10


def _sc_body(x_hbm, t_hbm, out_hbm, *rest):
    xbs = tuple(tuple(rest[st * _B + b] for b in range(_B)) for st in range(_ST))
    tbs = tuple(rest[12:15])
    sis = tuple(rest[15:18])
    sos = tuple(rest[18:21])
    sts = tuple(rest[21:24])

    wid = lax.axis_index("s") * _NC + lax.axis_index("c")
    base = wid * _ROWS

    def start_in(j, st):
        for b in range(_B):
            pltpu.async_copy(
                x_hbm.at[b, pl.ds(base + j * _R, _R), :], xbs[st][b], sis[st])

    def wait_in(st):
        for b in range(_B):
            pltpu.make_async_copy(
                x_hbm.at[0, pl.ds(base, _R), :], xbs[st][b], sis[st]).wait()

    def start_t(j, ti):
        pltpu.async_copy(
            t_hbm.at[pl.ds(base + j * _R, _R), :], tbs[ti], sts[ti])

    def wait_t(ti):
        pltpu.make_async_copy(
            t_hbm.at[pl.ds(base, _R), :], tbs[ti], sts[ti]).wait()

    def start_out(j, st):
        for b in range(_B):
            pltpu.async_copy(
                xbs[st][b], out_hbm.at[b, pl.ds(base + j * _R, _R), :],
                sos[st])

    def wait_out(st):
        for b in range(_B):
            pltpu.make_async_copy(
                xbs[st][b], out_hbm.at[0, pl.ds(base, _R), :], sos[st]).wait()

    def compute(st, ti):
        def _add(u, c):
            s = pl.ds(u * _L, _L)
            for r in range(_R):
                tv = tbs[ti][r, s]
                for b in range(_B):
                    xbs[st][b][r, s] = xbs[st][b][r, s] + tv
            return c

        lax.fori_loop(0, _D // _L, _add, 0)

    start_t(0, 0)
    start_in(0, 0)
    start_in(1, 1)

    def outer(g, carry):
        for jj in range(_ST):
            j = 3 * g + jj
            wait_in(jj)
            wait_t(jj)
            start_t(j + 1, (jj + 1) % 3)
            compute(jj, jj)
            start_out(j, jj)
            if jj == 0:
                @pl.when(g > 0)
                def _():
                    wait_out((jj + 2) % 3)
            else:
                wait_out((jj + 2) % 3)
            start_in(j + 2, (jj + 2) % 3)
        return carry

    lax.fori_loop(0, _NG, outer, 0)

    wait_in(0)
    wait_t(0)
    start_t(31, 1)
    compute(0, 0)
    start_out(30, 0)
    wait_out(2)

    wait_in(1)
    wait_t(1)
    compute(1, 1)
    start_out(31, 1)
    wait_out(0)
    wait_out(1)


@jax.jit
def _sc_add(x, table):
    mesh = plsc.VectorSubcoreMesh(core_axis_name="c", subcore_axis_name="s")
    f = functools.partial(
        pl.kernel,
        mesh=mesh,
        out_type=jax.ShapeDtypeStruct((_B, _S, _D), jnp.float32),
        scratch_types=(
            [pltpu.VMEM((_R, _D), jnp.float32)] * (_ST * _B)
            + [pltpu.VMEM((_R, _D), jnp.float32)] * _ST
            + [pltpu.SemaphoreType.DMA] * (3 * _ST)
        ),
    )(_sc_body)
    return f(x, table)


def kernel(x, table):
    return _sc_add(x, table)
